# baseline (speedup 1.0000x reference)
import functools

import jax
import jax.numpy as jnp
from jax import lax
from jax.experimental import pallas as pl
from jax.experimental.pallas import tpu as pltpu
from jax.experimental.pallas import tpu_sc as plsc

V_DIM = 1000000
D = 64
B = 16384
L = 20
LANES = 16
NC, NS = 2, 16
NW = NC * NS
BPW = B // NW
BC = 32
NCHUNK = BPW // BC
RPC = BC * L
GROUP = 128
NGROUP = RPC // GROUP
NWIN = RPC // LANES


def _sc_dots(center, pos_flat, neg_flat, w_center, w_context):
    mesh = plsc.VectorSubcoreMesh(
        core_axis_name="c", subcore_axis_name="s",
        num_cores=NC, num_subcores=NS)
    out_t = (jax.ShapeDtypeStruct((B * L,), jnp.float32),
             jax.ShapeDtypeStruct((B * L,), jnp.float32))
    scratch = [
        pltpu.VMEM((BC,), jnp.int32),
        pltpu.VMEM((RPC,), jnp.int32),
        pltpu.VMEM((RPC,), jnp.int32),
        pltpu.VMEM((BC, D), jnp.float32),
        pltpu.VMEM((RPC, D), jnp.float32),
        pltpu.VMEM((RPC, D), jnp.float32),
        pltpu.VMEM((RPC,), jnp.float32),
        pltpu.VMEM((RPC,), jnp.float32),
        pltpu.SemaphoreType.DMA,
    ]

    @functools.partial(pl.kernel, out_type=out_t, mesh=mesh,
                       scratch_types=scratch,
                       compiler_params=pltpu.CompilerParams(
                           use_tc_tiling_on_sc=False,
                           needs_layout_passes=False))
    def k(center_h, pos_h, neg_h, wcen_h, wctx_h, pdots_h, ndots_h,
          cidx_v, pidx_v, nidx_v, c_v, p_v, n_v, ps_v, ns_v, sem):
        wid = lax.axis_index("s") * NC + lax.axis_index("c")

        def chunk_body(t, carry):
            b0 = wid * BPW + t * BC
            r0 = b0 * L
            pltpu.sync_copy(center_h.at[pl.ds(b0, BC)], cidx_v)
            pltpu.sync_copy(pos_h.at[pl.ds(r0, RPC)], pidx_v)
            pltpu.sync_copy(neg_h.at[pl.ds(r0, RPC)], nidx_v)
            pltpu.async_copy(wcen_h.at[cidx_v], c_v, sem).wait()
            for g in range(NGROUP):
                pltpu.async_copy(
                    wctx_h.at[pidx_v.at[pl.ds(g * GROUP, GROUP)]],
                    p_v.at[pl.ds(g * GROUP, GROUP), :], sem).wait()
                pltpu.async_copy(
                    wctx_h.at[nidx_v.at[pl.ds(g * GROUP, GROUP)]],
                    n_v.at[pl.ds(g * GROUP, GROUP), :], sem).wait()

            def table(rows_v, idx_v, out_v):
                def win(w, c2):
                    row0 = w * LANES
                    rvec = row0 + lax.iota(jnp.int32, LANES)
                    bvec = rvec // L

                    def dstep(dd, acc):
                        dv = jnp.full((LANES,), dd, jnp.int32)
                        pv = plsc.load_gather(rows_v, [rvec, dv])
                        cv = plsc.load_gather(c_v, [bvec, dv])
                        return acc + pv * cv

                    acc = lax.fori_loop(0, D, dstep,
                                        jnp.zeros((LANES,), jnp.float32),
                                        unroll=8)
                    m = ((idx_v[pl.ds(row0, LANES)] != 0)
                         & (plsc.load_gather(cidx_v, [bvec]) != 0))
                    out_v[pl.ds(row0, LANES)] = jnp.where(m, acc, 0.0)
                    return c2

                lax.fori_loop(0, NWIN, win, 0)

            table(p_v, pidx_v, ps_v)
            table(n_v, nidx_v, ns_v)
            pltpu.sync_copy(ps_v, pdots_h.at[pl.ds(r0, RPC)])
            pltpu.sync_copy(ns_v, ndots_h.at[pl.ds(r0, RPC)])
            return carry

        lax.fori_loop(0, NCHUNK, chunk_body, 0)

    return k(center, pos_flat, neg_flat, w_center, w_context)


_ROWS, _COLS = 640, 512
_BLK = 64


def _tc_loss(pdots, ndots):
    def body(p_ref, n_ref, o_ref):
        i = pl.program_id(0)

        @pl.when(i == 0)
        def _init():
            o_ref[0, 0] = 0.0

        def ls(x):
            return jnp.minimum(x, 0.0) - jnp.log(1.0 + jnp.exp(-jnp.abs(x)))

        o_ref[0, 0] += jnp.sum(ls(p_ref[...])) + jnp.sum(ls(-n_ref[...]))

        @pl.when(i == pl.num_programs(0) - 1)
        def _fin():
            o_ref[0, 0] = o_ref[0, 0] * (-1.0 / B)

    out = pl.pallas_call(
        body,
        grid=(_ROWS // _BLK,),
        in_specs=[pl.BlockSpec((_BLK, _COLS), lambda i: (i, 0)),
                  pl.BlockSpec((_BLK, _COLS), lambda i: (i, 0))],
        out_specs=pl.BlockSpec(memory_space=pltpu.SMEM),
        out_shape=jax.ShapeDtypeStruct((1, 1), jnp.float32),
    )(pdots.reshape(_ROWS, _COLS), ndots.reshape(_ROWS, _COLS))
    return out[0, 0]


def kernel(center, pos, neg, W_center, W_context):
    pdots, ndots = _sc_dots(center, pos.reshape(-1), neg.reshape(-1),
                            W_center, W_context)
    return _tc_loss(pdots, ndots)


# speedup vs baseline: 1.5655x; 1.5655x over previous
import functools

import jax
import jax.numpy as jnp
from jax import lax
from jax.experimental import pallas as pl
from jax.experimental.pallas import tpu as pltpu
from jax.experimental.pallas import tpu_sc as plsc

V_DIM = 1000000
D = 64
B = 16384
L = 20
LANES = 16
NC, NS = 2, 16
NW = NC * NS
BPW = B // NW
BC = 32
NCHUNK = BPW // BC
RPC = BC * L
GROUP = 128
NGROUP = RPC // GROUP
NWIN = RPC // LANES


def _sc_dots(center, pos_flat, neg_flat, w_center, w_context):
    mesh = plsc.VectorSubcoreMesh(
        core_axis_name="c", subcore_axis_name="s",
        num_cores=NC, num_subcores=NS)
    out_t = (jax.ShapeDtypeStruct((B * L,), jnp.float32),
             jax.ShapeDtypeStruct((B * L,), jnp.float32))
    scratch = [
        pltpu.VMEM((BC,), jnp.int32),
        pltpu.VMEM((RPC,), jnp.int32),
        pltpu.VMEM((RPC,), jnp.int32),
        pltpu.VMEM((BC, D), jnp.float32),
        pltpu.VMEM((RPC, D), jnp.float32),
        pltpu.VMEM((RPC, D), jnp.float32),
        pltpu.VMEM((RPC,), jnp.float32),
        pltpu.VMEM((RPC,), jnp.float32),
        pltpu.VMEM((LANES * LANES,), jnp.float32),
        pltpu.SemaphoreType.DMA,
        pltpu.SemaphoreType.DMA,
    ]

    @functools.partial(pl.kernel, out_type=out_t, mesh=mesh,
                       scratch_types=scratch,
                       compiler_params=pltpu.CompilerParams(
                           use_tc_tiling_on_sc=False,
                           needs_layout_passes=False))
    def k(center_h, pos_h, neg_h, wcen_h, wctx_h, pdots_h, ndots_h,
          cidx_v, pidx_v, nidx_v, c_v, p_v, n_v, ps_v, ns_v, tmp_v,
          sem_a, sem_b):
        wid = lax.axis_index("s") * NC + lax.axis_index("c")
        iota = lax.iota(jnp.int32, LANES)
        scat_base = iota * LANES

        def table(rows_v, idx_v, out_v):
            def b_body(b, carry):
                c0 = c_v[b, pl.ds(0, LANES)]
                c1 = c_v[b, pl.ds(LANES, LANES)]
                c2 = c_v[b, pl.ds(2 * LANES, LANES)]
                c3 = c_v[b, pl.ds(3 * LANES, LANES)]
                for l in range(L):
                    r = b * L + l
                    a = (rows_v[r, pl.ds(0, LANES)] * c0
                         + rows_v[r, pl.ds(LANES, LANES)] * c1
                         + rows_v[r, pl.ds(2 * LANES, LANES)] * c2
                         + rows_v[r, pl.ds(3 * LANES, LANES)] * c3)
                    col = lax.rem(r, LANES)
                    plsc.store_scatter(tmp_v, [scat_base + col], a)

                    @pl.when(col == LANES - 1)
                    def _flush():
                        w0 = r - (LANES - 1)
                        s = tmp_v[pl.ds(0, LANES)]
                        for kk in range(1, LANES):
                            s = s + tmp_v[pl.ds(kk * LANES, LANES)]
                        rvec = w0 + iota
                        bvec = rvec // L
                        m = ((idx_v[pl.ds(w0, LANES)] != 0)
                             & (plsc.load_gather(cidx_v, [bvec]) != 0))
                        out_v[pl.ds(w0, LANES)] = jnp.where(m, s, 0.0)

                return carry

            lax.fori_loop(0, BC, b_body, 0)

        def chunk_body(t, carry):
            b0 = wid * BPW + t * BC
            r0 = b0 * L
            pltpu.sync_copy(center_h.at[pl.ds(b0, BC)], cidx_v)
            pltpu.sync_copy(pos_h.at[pl.ds(r0, RPC)], pidx_v)
            pltpu.sync_copy(neg_h.at[pl.ds(r0, RPC)], nidx_v)
            cp = pltpu.async_copy(wcen_h.at[cidx_v], c_v, sem_a)
            pcs = [pltpu.async_copy(
                wctx_h.at[pidx_v.at[pl.ds(g * GROUP, GROUP)]],
                p_v.at[pl.ds(g * GROUP, GROUP), :], sem_a)
                for g in range(NGROUP)]
            ncs = [pltpu.async_copy(
                wctx_h.at[nidx_v.at[pl.ds(g * GROUP, GROUP)]],
                n_v.at[pl.ds(g * GROUP, GROUP), :], sem_b)
                for g in range(NGROUP)]
            cp.wait()
            for c in pcs:
                c.wait()
            table(p_v, pidx_v, ps_v)
            for c in ncs:
                c.wait()
            table(n_v, nidx_v, ns_v)
            pltpu.sync_copy(ps_v, pdots_h.at[pl.ds(r0, RPC)])
            pltpu.sync_copy(ns_v, ndots_h.at[pl.ds(r0, RPC)])
            return carry

        lax.fori_loop(0, NCHUNK, chunk_body, 0)

    return k(center, pos_flat, neg_flat, w_center, w_context)


_ROWS, _COLS = 640, 512
_BLK = 64


def _tc_loss(pdots, ndots):
    def body(p_ref, n_ref, o_ref):
        i = pl.program_id(0)

        @pl.when(i == 0)
        def _init():
            o_ref[0, 0] = 0.0

        def ls(x):
            return jnp.minimum(x, 0.0) - jnp.log(1.0 + jnp.exp(-jnp.abs(x)))

        o_ref[0, 0] += jnp.sum(ls(p_ref[...])) + jnp.sum(ls(-n_ref[...]))

        @pl.when(i == pl.num_programs(0) - 1)
        def _fin():
            o_ref[0, 0] = o_ref[0, 0] * (-1.0 / B)

    out = pl.pallas_call(
        body,
        grid=(_ROWS // _BLK,),
        in_specs=[pl.BlockSpec((_BLK, _COLS), lambda i: (i, 0)),
                  pl.BlockSpec((_BLK, _COLS), lambda i: (i, 0))],
        out_specs=pl.BlockSpec(memory_space=pltpu.SMEM),
        out_shape=jax.ShapeDtypeStruct((1, 1), jnp.float32),
    )(pdots.reshape(_ROWS, _COLS), ndots.reshape(_ROWS, _COLS))
    return out[0, 0]


def kernel(center, pos, neg, W_center, W_context):
    pdots, ndots = _sc_dots(center, pos.reshape(-1), neg.reshape(-1),
                            W_center, W_context)
    return _tc_loss(pdots, ndots)
